# baseline (speedup 1.0000x reference)
import functools

import jax
import jax.numpy as jnp
import numpy as np
from jax import lax
from jax.experimental import pallas as pl
from jax.experimental.pallas import tpu as pltpu
from jax.experimental.pallas import tpu_sc as plsc

LOG2 = float(np.log(2.0))
_PREC = lax.Precision.HIGHEST


def _ssp(v):
    return jax.nn.softplus(v) - LOG2


def _filter_body(f_ref, r_ref, wf1t_ref, bf1_ref, wf2t_ref, bf2_ref, w_ref):
    h = jnp.dot(f_ref[...], wf1t_ref[...], preferred_element_type=jnp.float32,
                precision=_PREC) + bf1_ref[...]
    h = _ssp(h)
    w = jnp.dot(h, wf2t_ref[...], preferred_element_type=jnp.float32,
                precision=_PREC) + bf2_ref[...]
    r = r_ref[...]
    c = 0.5 * (jnp.cos(r * np.pi) + 1.0) * (r < 1.0).astype(jnp.float32)
    w_ref[...] = w * c


def _edge_filters(f_ij, r_ij, Wf1, bf1, Wf2, bf2, BE=2000):
    E, G = f_ij.shape
    F = Wf1.shape[0]
    grid = (E // BE,)
    return pl.pallas_call(
        _filter_body,
        grid=grid,
        in_specs=[
            pl.BlockSpec((BE, G), lambda i: (i, 0)),
            pl.BlockSpec((BE, 1), lambda i: (i, 0)),
            pl.BlockSpec((G, F), lambda i: (0, 0)),
            pl.BlockSpec((1, F), lambda i: (0, 0)),
            pl.BlockSpec((F, F), lambda i: (0, 0)),
            pl.BlockSpec((1, F), lambda i: (0, 0)),
        ],
        out_specs=pl.BlockSpec((BE, F), lambda i: (i, 0)),
        out_shape=jax.ShapeDtypeStruct((E, F), jnp.float32),
    )(f_ij, r_ij.reshape(E, 1), Wf1.T, bf1.reshape(1, F), Wf2.T,
      bf2.reshape(1, F))


def _linear_body(x_ref, wt_ref, o_ref):
    o_ref[...] = jnp.dot(x_ref[...], wt_ref[...],
                         preferred_element_type=jnp.float32, precision=_PREC)


def _node_linear(x, W, BN=2000):
    N, D = x.shape
    F = W.shape[0]
    return pl.pallas_call(
        _linear_body,
        grid=(N // BN,),
        in_specs=[
            pl.BlockSpec((BN, D), lambda i: (i, 0)),
            pl.BlockSpec((D, F), lambda i: (0, 0)),
        ],
        out_specs=pl.BlockSpec((BN, F), lambda i: (i, 0)),
        out_shape=jax.ShapeDtypeStruct((N, F), jnp.float32),
    )(x, W.T)


def _head_body(a_ref, wt_ref, b_ref, o_ref):
    a = a_ref[0] + a_ref[1]
    o_ref[...] = _ssp(jnp.dot(a, wt_ref[...],
                              preferred_element_type=jnp.float32,
                              precision=_PREC) + b_ref[...])


def _out_head(agg2, W_out, b_out, BN=2048):
    _, Np, F = agg2.shape
    O = W_out.shape[0]
    return pl.pallas_call(
        _head_body,
        grid=(Np // BN,),
        in_specs=[
            pl.BlockSpec((2, BN, F), lambda i: (0, i, 0)),
            pl.BlockSpec((F, O), lambda i: (0, 0)),
            pl.BlockSpec((1, O), lambda i: (0, 0)),
        ],
        out_specs=pl.BlockSpec((BN, O), lambda i: (i, 0)),
        out_shape=jax.ShapeDtypeStruct((Np, O), jnp.float32),
    )(agg2, W_out.T, b_out.reshape(1, O))


def _make_sc_agg(N, Np, E, F, K=80):
    NC, NS = 2, 16
    NW = NC * NS
    RPT = Np // NS
    EPW = E // NW
    n_chunks = EPW // K
    mesh = plsc.VectorSubcoreMesh(core_axis_name="c", subcore_axis_name="s")

    @functools.partial(
        pl.kernel,
        out_type=jax.ShapeDtypeStruct((NC, Np, F), jnp.float32),
        mesh=mesh,
        scratch_types=[
            pltpu.VMEM_SHARED((Np, F), jnp.float32),
            pltpu.VMEM((K,), jnp.int32),
            pltpu.VMEM((K,), jnp.int32),
            pltpu.VMEM((K, F), jnp.float32),
            pltpu.VMEM((K, F), jnp.float32),
            pltpu.SemaphoreType.DMA,
        ],
    )
    def sc_agg(xf_hbm, w_hbm, indi_hbm, indj_hbm, z_hbm, out_hbm,
               agg_sp, idxj_v, idxi_v, rows_v, w_v, sem):
        c = lax.axis_index("c")
        s = lax.axis_index("s")
        row0 = s * RPT
        pltpu.sync_copy(z_hbm.at[pl.ds(row0, RPT)],
                        agg_sp.at[pl.ds(row0, RPT)])
        plsc.subcore_barrier()

        ebase = (s * NC + c) * EPW

        def chunk(k, _):
            off = ebase + k * K
            pltpu.sync_copy(indj_hbm.at[pl.ds(off, K)], idxj_v)
            pltpu.sync_copy(indi_hbm.at[pl.ds(off, K)], idxi_v)
            pltpu.async_copy(xf_hbm.at[idxj_v], rows_v, sem).wait()
            pltpu.sync_copy(w_hbm.at[pl.ds(off, K)], w_v)

            def mul(e, _):
                for d in range(F // 16):
                    sl = pl.ds(d * 16, 16)
                    rows_v[e, sl] = rows_v[e, sl] * w_v[e, sl]
                return 0

            lax.fori_loop(0, K, mul, 0, unroll=4)
            pltpu.sync_copy(rows_v, agg_sp.at[idxi_v], add=True)
            return 0

        lax.fori_loop(0, n_chunks, chunk, 0)
        plsc.subcore_barrier()
        pltpu.sync_copy(agg_sp.at[pl.ds(row0, RPT)],
                        out_hbm.at[c, pl.ds(row0, RPT)])

    return sc_agg


def kernel(x, r_ij, f_ij, ind_i, ind_j, W_in2f, Wf1, bf1, Wf2, bf2,
           W_out, b_out):
    N, D = x.shape
    E = ind_i.shape[0]
    F = Wf1.shape[0]
    Np = 10240 if N == 10000 else ((N + 1023) // 1024) * 1024
    W = _edge_filters(f_ij, r_ij, Wf1, bf1, Wf2, bf2)
    xf = _node_linear(x, W_in2f)
    z = jnp.zeros((Np, F), jnp.float32)
    sc = _make_sc_agg(N, Np, E, F)
    agg2 = sc(xf, W, ind_i.astype(jnp.int32), ind_j.astype(jnp.int32), z)
    return _out_head(agg2, W_out, b_out)[:N]


# speedup vs baseline: 1.3123x; 1.3123x over previous
import functools

import jax
import jax.numpy as jnp
import numpy as np
from jax import lax
from jax.experimental import pallas as pl
from jax.experimental.pallas import tpu as pltpu
from jax.experimental.pallas import tpu_sc as plsc

LOG2 = float(np.log(2.0))
_PREC = lax.Precision.HIGHEST


def _ssp(v):
    return jax.nn.softplus(v) - LOG2


def _make_filter_body(BE):
    def _filter_body(f_ref, r_ref, wf1t_ref, bf1_ref, wf2t_ref, bf2_ref,
                     w_ref, c_ref):
        h = jnp.dot(f_ref[...], wf1t_ref[...],
                    preferred_element_type=jnp.float32,
                    precision=_PREC) + bf1_ref[...]
        h = _ssp(h)
        w_ref[...] = jnp.dot(h, wf2t_ref[...],
                             preferred_element_type=jnp.float32) + bf2_ref[...]
        r = r_ref[...]
        c = 0.5 * (jnp.cos(r * np.pi) + 1.0) * (r < 1.0).astype(jnp.float32)
        c_ref[...] = c.reshape(BE)
    return _filter_body


def _edge_filters(f_ij, r_ij, Wf1, bf1, Wf2, bf2, BE=512):
    E, G = f_ij.shape
    F = Wf1.shape[0]
    grid = (E // BE,)
    return pl.pallas_call(
        _make_filter_body(BE),
        grid=grid,
        in_specs=[
            pl.BlockSpec((BE, G), lambda i: (i, 0)),
            pl.BlockSpec((1, BE // 128, 128), lambda i: (i, 0, 0)),
            pl.BlockSpec((G, F), lambda i: (0, 0)),
            pl.BlockSpec((1, F), lambda i: (0, 0)),
            pl.BlockSpec((F, F), lambda i: (0, 0)),
            pl.BlockSpec((1, F), lambda i: (0, 0)),
        ],
        out_specs=[
            pl.BlockSpec((BE, F), lambda i: (i, 0)),
            pl.BlockSpec((BE,), lambda i: (i,)),
        ],
        out_shape=[
            jax.ShapeDtypeStruct((E, F), jnp.float32),
            jax.ShapeDtypeStruct((E,), jnp.float32),
        ],
    )(f_ij, r_ij.reshape(E // BE, BE // 128, 128), Wf1.T, bf1.reshape(1, F),
      Wf2.T, bf2.reshape(1, F))


def _linear_body(x_ref, wt_ref, o_ref):
    o_ref[...] = jnp.dot(x_ref[...], wt_ref[...],
                         preferred_element_type=jnp.float32, precision=_PREC)


def _node_linear(x, W, BN=2000):
    N, D = x.shape
    F = W.shape[0]
    return pl.pallas_call(
        _linear_body,
        grid=(N // BN,),
        in_specs=[
            pl.BlockSpec((BN, D), lambda i: (i, 0)),
            pl.BlockSpec((D, F), lambda i: (0, 0)),
        ],
        out_specs=pl.BlockSpec((BN, F), lambda i: (i, 0)),
        out_shape=jax.ShapeDtypeStruct((N, F), jnp.float32),
    )(x, W.T)


def _head_body(a_ref, wt_ref, b_ref, o_ref):
    a = a_ref[0] + a_ref[1]
    o_ref[...] = _ssp(jnp.dot(a, wt_ref[...],
                              preferred_element_type=jnp.float32,
                              precision=_PREC) + b_ref[...])


def _out_head(agg2, W_out, b_out, BN=2048):
    _, Np, F = agg2.shape
    O = W_out.shape[0]
    return pl.pallas_call(
        _head_body,
        grid=(Np // BN,),
        in_specs=[
            pl.BlockSpec((2, BN, F), lambda i: (0, i, 0)),
            pl.BlockSpec((F, O), lambda i: (0, 0)),
            pl.BlockSpec((1, O), lambda i: (0, 0)),
        ],
        out_specs=pl.BlockSpec((BN, O), lambda i: (i, 0)),
        out_shape=jax.ShapeDtypeStruct((Np, O), jnp.float32),
    )(agg2, W_out.T, b_out.reshape(1, O))


def _make_sc_agg(N, Np, E, F, K=80):
    NC, NS = 2, 16
    NW = NC * NS
    RPT = Np // NS
    EPW = E // NW
    n_chunks = EPW // K
    mesh = plsc.VectorSubcoreMesh(core_axis_name="c", subcore_axis_name="s")

    @functools.partial(
        pl.kernel,
        out_type=jax.ShapeDtypeStruct((NC, Np, F), jnp.float32),
        mesh=mesh,
        compiler_params=pltpu.CompilerParams(needs_layout_passes=False),
        scratch_types=[
            pltpu.VMEM_SHARED((Np, F), jnp.float32),
            pltpu.VMEM((K,), jnp.int32),
            pltpu.VMEM((K,), jnp.int32),
            pltpu.VMEM((K, F), jnp.float32),
            pltpu.VMEM((K, F), jnp.float32),
            pltpu.VMEM((K,), jnp.float32),
            pltpu.SemaphoreType.DMA,
        ],
    )
    def sc_agg(xf_hbm, w_hbm, c_hbm, indi_hbm, indj_hbm, z_hbm, out_hbm,
               agg_sp, idxj_v, idxi_v, rows_v, w_v, c_v, sem):
        c = lax.axis_index("c")
        s = lax.axis_index("s")
        row0 = s * RPT
        pltpu.sync_copy(z_hbm.at[pl.ds(row0, RPT)],
                        agg_sp.at[pl.ds(row0, RPT)])
        plsc.subcore_barrier()

        ebase = (s * NC + c) * EPW

        def chunk(k, _):
            off = ebase + k * K
            pltpu.sync_copy(indj_hbm.at[pl.ds(off, K)], idxj_v)
            pltpu.sync_copy(indi_hbm.at[pl.ds(off, K)], idxi_v)
            pltpu.sync_copy(c_hbm.at[pl.ds(off, K)], c_v)
            pltpu.async_copy(xf_hbm.at[idxj_v], rows_v, sem).wait()
            pltpu.sync_copy(w_hbm.at[pl.ds(off, K)], w_v)

            def mul(e, _):
                cb = plsc.load_gather(c_v, [jnp.full((16,), e, jnp.int32)])
                for d in range(F // 16):
                    sl = pl.ds(d * 16, 16)
                    rows_v[e, sl] = rows_v[e, sl] * w_v[e, sl] * cb
                return 0

            lax.fori_loop(0, K, mul, 0, unroll=4)
            pltpu.sync_copy(rows_v, agg_sp.at[idxi_v], add=True)
            return 0

        lax.fori_loop(0, n_chunks, chunk, 0)
        plsc.subcore_barrier()
        pltpu.sync_copy(agg_sp.at[pl.ds(row0, RPT)],
                        out_hbm.at[c, pl.ds(row0, RPT)])

    return sc_agg


def kernel(x, r_ij, f_ij, ind_i, ind_j, W_in2f, Wf1, bf1, Wf2, bf2,
           W_out, b_out):
    N, D = x.shape
    E = ind_i.shape[0]
    F = Wf1.shape[0]
    Np = 10240 if N == 10000 else ((N + 1023) // 1024) * 1024
    W, C = _edge_filters(f_ij, r_ij, Wf1, bf1, Wf2, bf2)
    xf = _node_linear(x, W_in2f)
    z = jnp.zeros((Np, F), jnp.float32)
    sc = _make_sc_agg(N, Np, E, F)
    agg2 = sc(xf, W, C, ind_i.astype(jnp.int32), ind_j.astype(jnp.int32), z)
    return _out_head(agg2, W_out, b_out)[:N]
